# baseline (speedup 1.0000x reference)
import functools

import jax
import jax.numpy as jnp
from jax import lax
from jax.experimental import pallas as pl
from jax.experimental.pallas import tpu as pltpu
from jax.experimental.pallas import tpu_sc as plsc

_NF = 8
_SUB = 16
_C = 256
_V = _C * _NF
_G = _V // 128
_BPC = _C // 128


def _worker_id(NC):
    return lax.axis_index("s") * NC + lax.axis_index("c")


def _pipeline(my_n, fire_load, do_chunk, drain_writes):
    @pl.when(my_n >= 1)
    def _():
        fire_load(0, 0)

    def pair(tt, c):
        t0 = 2 * tt

        def full_chunk(t, p):
            @pl.when(t + 1 < my_n)
            def _():
                fire_load(t + 1, 1 - p)

            @pl.when(t >= 2)
            def _():
                drain_writes(p)

            do_chunk(t, p)

        full_chunk(t0, 0)

        @pl.when(t0 + 1 < my_n)
        def _():
            full_chunk(t0 + 1, 1)
        return c

    lax.fori_loop(0, (my_n + 1) // 2, pair, 0)

    for p in range(2):
        outstanding = ((my_n >= 1) & ((my_n - 1) % 2 == p)) | (
            (my_n >= 2) & (my_n % 2 == p))

        @pl.when(outstanding)
        def _(p=p):
            drain_writes(p)


def _gather_groups(p, tab, idx, rows, semg, compute_group, write_group):
    gcopies = []
    for g in range(_G):
        compute_group(p, g)
        gcopies.append(pltpu.async_copy(
            tab.at[idx.at[p, g]],
            rows.at[p, pl.ds(g * 128, 128)],
            semg,
        ))
    for g, cp in enumerate(gcopies):
        cp.wait()
        write_group(p, g)


_SCRATCH_COMMON = [
    pltpu.VMEM((2, _G, 128), jnp.int32),
    pltpu.VMEM((2, _V, _SUB), jnp.float32),
    pltpu.SemaphoreType.DMA,
    pltpu.SemaphoreType.DMA,
    pltpu.SemaphoreType.DMA,
    pltpu.SemaphoreType.DMA,
    pltpu.SemaphoreType.DMA,
]


@functools.lru_cache(maxsize=None)
def _build_node(N, K):
    info = plsc.get_sparse_core_info()
    NC, NW = info.num_cores, info.num_cores * info.num_subcores
    mesh = plsc.VectorSubcoreMesh(core_axis_name="c", subcore_axis_name="s")

    @functools.partial(
        pl.kernel,
        mesh=mesh,
        compiler_params=pltpu.CompilerParams(use_tc_tiling_on_sc=False),
        out_type=jax.ShapeDtypeStruct((N * _NF, _SUB), jnp.float32),
        scratch_types=[pltpu.VMEM((2, _V), jnp.float32)] + _SCRATCH_COMMON,
    )
    def tok_node(xa, ntab, out_x,
                 attr_f, idx, rows, sema0, sema1, semg, semw0, semw1):
        wid = _worker_id(NC)
        iota = lax.broadcasted_iota(jnp.int32, (16,), 0)
        field = jnp.bitwise_and(iota, _NF - 1)
        sema = (sema0, sema1)
        semw = (semw0, semw1)

        n_chunks = (N + _C - 1) // _C
        my_n = (n_chunks - wid + NW - 1) // NW

        def chunk_base(t):
            return jnp.minimum((wid + t * NW) * _C, N - _C) * _NF

        def fire_load(t, p):
            pltpu.async_copy(
                xa.at[pl.ds(chunk_base(t), _V)], attr_f.at[p], sema[p])

        def drain_writes(p):
            pltpu.make_async_copy(
                ntab.at[pl.ds(0, _V)], rows.at[p], semw[p]).wait()

        def compute_group(p, g):
            def jbody(j, c):
                col = attr_f[p, pl.ds(g * 128 + j * 16, 16)]
                v = (col * float(K)).astype(jnp.int32) + 1
                v = jnp.minimum(jnp.maximum(v, 0), K - 1)
                idx[p, g, pl.ds(j * 16, 16)] = v * _NF + field
                return c
            lax.fori_loop(0, 8, jbody, 0)

        def do_chunk(t, p):
            pltpu.make_async_copy(
                xa.at[pl.ds(0, _V)], attr_f.at[p], sema[p]).wait()
            base = chunk_base(t)

            def write_group(p, g):
                pltpu.async_copy(
                    rows.at[p, pl.ds(g * 128, 128)],
                    out_x.at[pl.ds(base + g * 128, 128)],
                    semw[p],
                )

            _gather_groups(p, ntab, idx, rows, semg,
                           compute_group, write_group)

        _pipeline(my_n, fire_load, do_chunk, drain_writes)

    return tok_node


@functools.lru_cache(maxsize=None)
def _build_edge(E, K):
    assert E % _C == 0
    NB = E // 128
    info = plsc.get_sparse_core_info()
    NC, NW = info.num_cores, info.num_cores * info.num_subcores
    mesh = plsc.VectorSubcoreMesh(core_axis_name="c", subcore_axis_name="s")

    @functools.partial(
        pl.kernel,
        mesh=mesh,
        compiler_params=pltpu.CompilerParams(use_tc_tiling_on_sc=False),
        out_type=jax.ShapeDtypeStruct((E, _NF * _SUB), jnp.float32),
        scratch_types=(
            [pltpu.VMEM((2, _BPC, _NF, 128), jnp.float32)] + _SCRATCH_COMMON),
    )
    def tok_edge(ea, etab, out_e,
                 attr_b, idx, rows, sema0, sema1, semg, semw0, semw1):
        wid = _worker_id(NC)
        sema = (sema0, sema1)
        semw = (semw0, semw1)

        n_chunks = NB // _BPC
        my_n = (n_chunks - wid + NW - 1) // NW

        def chunk_blk(t):
            return (wid + t * NW) * _BPC

        def fire_load(t, p):
            pltpu.async_copy(
                ea.at[pl.ds(chunk_blk(t), _BPC)], attr_b.at[p], sema[p])

        def drain_writes(p):
            pltpu.make_async_copy(
                etab.at[pl.ds(0, _V)], rows.at[p], semw[p]).wait()

        def compute_group(p, g):
            bl, f = divmod(g, _NF)

            def jbody(j, c):
                col = attr_b[p, bl, f, pl.ds(j * 16, 16)]
                v = (col * float(K)).astype(jnp.int32) + 1
                v = jnp.minimum(jnp.maximum(v, 0), K - 1)
                idx[p, g, pl.ds(j * 16, 16)] = v * _NF + f
                return c
            lax.fori_loop(0, 8, jbody, 0)

        def do_chunk(t, p):
            pltpu.make_async_copy(
                ea.at[pl.ds(0, _BPC)], attr_b.at[p], sema[p]).wait()
            row0 = chunk_blk(t) * 128

            def write_group(p, g):
                bl, f = divmod(g, _NF)
                pltpu.async_copy(
                    rows.at[p, pl.ds(g * 128, 128)],
                    out_e.at[pl.ds(row0 + bl * 128, 128),
                             pl.ds(f * _SUB, _SUB)],
                    semw[p],
                )

            _gather_groups(p, etab, idx, rows, semg,
                           compute_group, write_group)

        _pipeline(my_n, fire_load, do_chunk, drain_writes)

    return tok_edge


@jax.jit
def kernel(x, edge_index, edge_attr, node_tables, edge_tables):
    del edge_index
    N = x.shape[0]
    E = edge_attr.shape[0]
    node_k = node_tables.shape[1]
    edge_k = edge_tables.shape[1]
    ea_blocked = edge_attr.reshape(E // 128, 128, _NF).transpose(0, 2, 1)

    def flat_table(tables, via_3d=False):
        K = tables.shape[1]
        wide = lax.optimization_barrier(
            jnp.swapaxes(tables, 1, 2).reshape(_NF * _SUB, -1))
        if via_3d:
            w = K // 4
            t = jnp.concatenate(
                [lax.optimization_barrier(wide[:, i * w:(i + 1) * w].T)
                 for i in range(4)], axis=0)
        else:
            t = wide.T
        return lax.optimization_barrier(t).reshape(-1, _SUB)

    out_e = _build_edge(E, edge_k)(ea_blocked, flat_table(edge_tables))
    out_x = _build_node(N, node_k)(
        x.reshape(-1), flat_table(node_tables, via_3d=True))
    return (out_x.reshape(N, _NF * _SUB), out_e)


# speedup vs baseline: 1.3986x; 1.3986x over previous
import functools

import jax
import jax.numpy as jnp
from jax import lax
from jax.experimental import pallas as pl
from jax.experimental.pallas import tpu as pltpu
from jax.experimental.pallas import tpu_sc as plsc

_NF = 8
_SUB = 16
_C = 256
_V = _C * _NF
_G = _V // 128
_BPC = _C // 128


def _worker_id(NC):
    return lax.axis_index("s") * NC + lax.axis_index("c")


def _pipeline(my_n, fire_load, do_chunk, drain_writes):
    @pl.when(my_n >= 1)
    def _():
        fire_load(0, 0)

    def pair(tt, c):
        t0 = 2 * tt

        def full_chunk(t, p):
            @pl.when(t + 1 < my_n)
            def _():
                fire_load(t + 1, 1 - p)

            @pl.when(t >= 2)
            def _():
                drain_writes(p)

            do_chunk(t, p)

        full_chunk(t0, 0)

        @pl.when(t0 + 1 < my_n)
        def _():
            full_chunk(t0 + 1, 1)
        return c

    lax.fori_loop(0, (my_n + 1) // 2, pair, 0)

    for p in range(2):
        outstanding = ((my_n >= 1) & ((my_n - 1) % 2 == p)) | (
            (my_n >= 2) & (my_n % 2 == p))

        @pl.when(outstanding)
        def _(p=p):
            drain_writes(p)


def _gather_groups(p, tab, idx, rows, semg, compute_group, write_group):
    gcopies = []
    for g in range(_G):
        compute_group(p, g)
        gcopies.append(pltpu.async_copy(
            tab.at[idx.at[p, g]],
            rows.at[p, pl.ds(g * 128, 128)],
            semg,
        ))
    for g, cp in enumerate(gcopies):
        cp.wait()
        write_group(p, g)


_SCRATCH_COMMON = [
    pltpu.VMEM((2, _G, 128), jnp.int32),
    pltpu.VMEM((2, _V, _SUB), jnp.float32),
    pltpu.SemaphoreType.DMA,
    pltpu.SemaphoreType.DMA,
    pltpu.SemaphoreType.DMA,
    pltpu.SemaphoreType.DMA,
    pltpu.SemaphoreType.DMA,
]


@functools.lru_cache(maxsize=None)
def _build_node(N, K):
    info = plsc.get_sparse_core_info()
    NC, NW = info.num_cores, info.num_cores * info.num_subcores
    mesh = plsc.VectorSubcoreMesh(core_axis_name="c", subcore_axis_name="s")

    @functools.partial(
        pl.kernel,
        mesh=mesh,
        compiler_params=pltpu.CompilerParams(use_tc_tiling_on_sc=False),
        out_type=jax.ShapeDtypeStruct((N * _NF, _SUB), jnp.float32),
        scratch_types=[pltpu.VMEM((2, _V), jnp.float32)] + _SCRATCH_COMMON,
    )
    def tok_node(xa, ntab, out_x,
                 attr_f, idx, rows, sema0, sema1, semg, semw0, semw1):
        wid = _worker_id(NC)
        iota = lax.broadcasted_iota(jnp.int32, (16,), 0)
        field = jnp.bitwise_and(iota, _NF - 1)
        sema = (sema0, sema1)
        semw = (semw0, semw1)

        n_chunks = (N + _C - 1) // _C
        my_n = (n_chunks - wid + NW - 1) // NW

        def chunk_base(t):
            return jnp.minimum((wid + t * NW) * _C, N - _C) * _NF

        def fire_load(t, p):
            pltpu.async_copy(
                xa.at[pl.ds(chunk_base(t), _V)], attr_f.at[p], sema[p])

        def drain_writes(p):
            pltpu.make_async_copy(
                ntab.at[pl.ds(0, _V)], rows.at[p], semw[p]).wait()

        def compute_group(p, g):
            def jbody(j, c):
                col = attr_f[p, pl.ds(g * 128 + j * 16, 16)]
                v = (col * float(K)).astype(jnp.int32) + 1
                v = jnp.minimum(jnp.maximum(v, 0), K - 1)
                idx[p, g, pl.ds(j * 16, 16)] = v * _NF + field
                return c
            lax.fori_loop(0, 8, jbody, 0)

        def do_chunk(t, p):
            pltpu.make_async_copy(
                xa.at[pl.ds(0, _V)], attr_f.at[p], sema[p]).wait()
            base = chunk_base(t)

            def write_group(p, g):
                pltpu.async_copy(
                    rows.at[p, pl.ds(g * 128, 128)],
                    out_x.at[pl.ds(base + g * 128, 128)],
                    semw[p],
                )

            _gather_groups(p, ntab, idx, rows, semg,
                           compute_group, write_group)

        _pipeline(my_n, fire_load, do_chunk, drain_writes)

    return tok_node


@functools.lru_cache(maxsize=None)
def _build_edge(E, K):
    assert E % _C == 0
    NB = E // 128
    info = plsc.get_sparse_core_info()
    NC, NW = info.num_cores, info.num_cores * info.num_subcores
    mesh = plsc.VectorSubcoreMesh(core_axis_name="c", subcore_axis_name="s")

    @functools.partial(
        pl.kernel,
        mesh=mesh,
        compiler_params=pltpu.CompilerParams(use_tc_tiling_on_sc=False),
        out_type=jax.ShapeDtypeStruct((E, _NF * _SUB), jnp.float32),
        scratch_types=(
            [pltpu.VMEM((2, _BPC, _NF, 128), jnp.float32)] + _SCRATCH_COMMON),
    )
    def tok_edge(ea, etab, out_e,
                 attr_b, idx, rows, sema0, sema1, semg, semw0, semw1):
        wid = _worker_id(NC)
        sema = (sema0, sema1)
        semw = (semw0, semw1)

        n_chunks = NB // _BPC
        my_n = (n_chunks - wid + NW - 1) // NW

        def chunk_blk(t):
            return (wid + t * NW) * _BPC

        def fire_load(t, p):
            pltpu.async_copy(
                ea.at[pl.ds(chunk_blk(t), _BPC)], attr_b.at[p], sema[p])

        def drain_writes(p):
            pltpu.make_async_copy(
                etab.at[pl.ds(0, _V)], rows.at[p], semw[p]).wait()

        def compute_group(p, g):
            bl, f = divmod(g, _NF)

            def jbody(j, c):
                col = attr_b[p, bl, f, pl.ds(j * 16, 16)]
                v = (col * float(K)).astype(jnp.int32) + 1
                v = jnp.minimum(jnp.maximum(v, 0), K - 1)
                idx[p, g, pl.ds(j * 16, 16)] = v * _NF + f
                return c
            lax.fori_loop(0, 8, jbody, 0)

        def do_chunk(t, p):
            pltpu.make_async_copy(
                ea.at[pl.ds(0, _BPC)], attr_b.at[p], sema[p]).wait()
            row0 = chunk_blk(t) * 128

            def write_group(p, g):
                bl, f = divmod(g, _NF)
                pltpu.async_copy(
                    rows.at[p, pl.ds(g * 128, 128)],
                    out_e.at[pl.ds(row0 + bl * 128, 128),
                             pl.ds(f * _SUB, _SUB)],
                    semw[p],
                )

            _gather_groups(p, etab, idx, rows, semg,
                           compute_group, write_group)

        _pipeline(my_n, fire_load, do_chunk, drain_writes)

    return tok_edge


@jax.jit
def kernel(x, edge_index, edge_attr, node_tables, edge_tables):
    del edge_index
    N = x.shape[0]
    E = edge_attr.shape[0]
    node_k = node_tables.shape[1]
    edge_k = edge_tables.shape[1]
    ea_blocked = edge_attr.reshape(E // 128, 128, _NF).transpose(0, 2, 1)

    def flat_table(tables):
        wide = lax.optimization_barrier(
            jnp.swapaxes(tables, 1, 2).reshape(_NF * _SUB, -1))
        return lax.optimization_barrier(wide.T).reshape(-1, _SUB)

    out_e = _build_edge(E, edge_k)(ea_blocked, flat_table(edge_tables))
    out_x = _build_node(N, node_k)(x.reshape(-1), flat_table(node_tables))
    return (out_x.reshape(N, _NF * _SUB), out_e)
